# baseline (speedup 1.0000x reference)
import functools

import jax
import jax.numpy as jnp
from jax import lax
from jax.experimental import pallas as pl
from jax.experimental.pallas import tpu as pltpu
from jax.experimental.pallas import tpu_sc as plsc

EMB_D = 64
PAD_D = 128
SCALE = float(EMB_D) ** 0.5
LANES = 16
NBUF = 2
CHUNK = 256


@functools.partial(jax.jit, static_argnames=("b_total", "h_total"))
def _lookup(x_flat, table_p, b_total, h_total):
    info = plsc.get_sparse_core_info()
    nw = info.num_cores * info.num_subcores
    b_per_w = b_total // nw
    sub_per_h = b_per_w // CHUNK
    n_chunks = h_total * sub_per_h
    assert b_per_w % CHUNK == 0 and b_total % nw == 0

    mesh = plsc.VectorSubcoreMesh(core_axis_name="c", subcore_axis_name="s")

    @functools.partial(
        pl.kernel,
        mesh=mesh,
        out_type=jax.ShapeDtypeStruct((h_total, EMB_D, b_total), jnp.float32),
        scratch_types=[
            [pltpu.VMEM((CHUNK,), jnp.int32) for _ in range(NBUF)],
            [pltpu.VMEM((CHUNK, PAD_D), jnp.float32) for _ in range(NBUF)],
            [pltpu.VMEM((8, 128), jnp.float32) for _ in range(16)],
            [pltpu.SemaphoreType.DMA for _ in range(NBUF)],
            pltpu.SemaphoreType.DMA,
            [pltpu.SemaphoreType.DMA for _ in range(NBUF)],
        ],
        compiler_params=pltpu.CompilerParams(
            use_tc_tiling_on_sc=True, needs_layout_passes=False
        ),
    )
    def k(x_hbm, table_hbm, out_hbm, idx_v, rows, tbuf, sem_g, sem_t, sem_i):
        wid = lax.axis_index("s") * info.num_cores + lax.axis_index("c")
        wb = wid * b_per_w
        biota = lax.iota(jnp.int32, LANES)
        bidx = [biota + g * LANES for g in range(CHUNK // LANES)]
        dcols = [jnp.full((LANES,), d, jnp.int32) for d in range(EMB_D)]

        def x_off(c):
            h = c // sub_per_h
            return h * b_total + wb + (c % sub_per_h) * CHUNK

        pltpu.async_copy(x_hbm.at[pl.ds(x_off(0), CHUNK)], idx_v[0], sem_i[0])
        pltpu.make_async_copy(
            x_hbm.at[pl.ds(x_off(0), CHUNK)], idx_v[0], sem_i[0]
        ).wait()
        pltpu.async_copy(table_hbm.at[idx_v[0]], rows[0], sem_g[0])
        pltpu.async_copy(x_hbm.at[pl.ds(x_off(1), CHUNK)], idx_v[1], sem_i[1])

        @pl.loop(0, n_chunks, step=NBUF)
        def _chunk_loop(c0):
            for b in range(NBUF):
                c = c0 + b
                nb = (b + 1) % NBUF
                nxt = c + 1
                h = c // sub_per_h
                bb = wb + (c % sub_per_h) * CHUNK

                @pl.when(nxt < n_chunks)
                def _issue_next_gather():
                    pltpu.make_async_copy(
                        x_hbm.at[pl.ds(x_off(nxt), CHUNK)], idx_v[nb], sem_i[nb]
                    ).wait()
                    pltpu.async_copy(
                        table_hbm.at[idx_v[nb]], rows[nb], sem_g[nb]
                    )

                pltpu.make_async_copy(
                    table_hbm.at[idx_v[b]], rows[b], sem_g[b]
                ).wait()

                @pl.when(c + 2 < n_chunks)
                def _prefetch_idx():
                    pltpu.async_copy(
                        x_hbm.at[pl.ds(x_off(c + 2), CHUNK)],
                        idx_v[b],
                        sem_i[b],
                    )

                @pl.when(c > 0)
                def _drain_prev_tiles():
                    for t in range(16):
                        i, j = divmod(t, 2)
                        pltpu.make_async_copy(
                            tbuf[t],
                            out_hbm.at[
                                h, pl.ds(8 * i, 8), pl.ds(bb + 128 * j, 128)
                            ],
                            sem_t,
                        ).wait()

                for t in range(16):
                    i, j = divmod(t, 2)
                    for s in range(8):
                        d = 8 * i + s
                        for g in range(8):
                            vals = plsc.load_gather(
                                rows[b], [bidx[8 * j + g], dcols[d]]
                            )
                            tbuf[t][s, pl.ds(g * LANES, LANES)] = vals * SCALE

                    pltpu.async_copy(
                        tbuf[t],
                        out_hbm.at[
                            h, pl.ds(8 * i, 8), pl.ds(bb + 128 * j, 128)
                        ],
                        sem_t,
                    )

        lastc = n_chunks - 1
        lh = lastc // sub_per_h
        lbb = wb + (lastc % sub_per_h) * CHUNK
        for t in range(16):
            i, j = divmod(t, 2)
            pltpu.make_async_copy(
                tbuf[t],
                out_hbm.at[lh, pl.ds(8 * i, 8), pl.ds(lbb + 128 * j, 128)],
                sem_t,
            ).wait()

    return k(x_flat, table_p)


def kernel(x, table):
    b, h = x.shape
    x_flat = x.T.reshape(-1)
    table_p = jnp.pad(table, ((0, 0), (0, PAD_D - EMB_D)))
    out = _lookup(x_flat, table_p, b, h)
    return out.transpose(2, 0, 1)


# speedup vs baseline: 1.8330x; 1.8330x over previous
import functools

import jax
import jax.numpy as jnp
from jax import lax
from jax.experimental import pallas as pl
from jax.experimental.pallas import tpu as pltpu
from jax.experimental.pallas import tpu_sc as plsc

EMB_D = 64
PAD_D = 128
SCALE = float(EMB_D) ** 0.5
LANES = 16
NBUF = 2
CHUNK = 256


@functools.partial(jax.jit, static_argnames=("b_total", "h_total"))
def _lookup(x_flat, table_p, b_total, h_total):
    info = plsc.get_sparse_core_info()
    nw = info.num_cores * info.num_subcores
    b_per_w = b_total // nw
    sub_per_h = b_per_w // CHUNK
    n_chunks = h_total * sub_per_h
    assert b_per_w % CHUNK == 0 and b_total % nw == 0

    mesh = plsc.VectorSubcoreMesh(core_axis_name="c", subcore_axis_name="s")

    @functools.partial(
        pl.kernel,
        mesh=mesh,
        out_type=jax.ShapeDtypeStruct((h_total, EMB_D, b_total), jnp.float32),
        scratch_types=[
            [pltpu.VMEM((CHUNK,), jnp.int32) for _ in range(NBUF)],
            [pltpu.VMEM((CHUNK, PAD_D), jnp.float32) for _ in range(NBUF)],
            pltpu.VMEM((16, 8, 128), jnp.float32),
            [pltpu.SemaphoreType.DMA for _ in range(NBUF)],
            pltpu.SemaphoreType.DMA,
            [pltpu.SemaphoreType.DMA for _ in range(NBUF)],
        ],
        compiler_params=pltpu.CompilerParams(
            use_tc_tiling_on_sc=True, needs_layout_passes=False
        ),
    )
    def k(x_hbm, table_hbm, out_hbm, idx_v, rows, tbuf, sem_g, sem_t, sem_i):
        wid = lax.axis_index("s") * info.num_cores + lax.axis_index("c")
        wb = wid * b_per_w
        biota = lax.iota(jnp.int32, LANES)
        e_k = [(biota + k) & 15 for k in range(LANES)]
        s_k = [e & 7 for e in e_k]
        r_k = [(e & 8) >> 2 for e in e_k]

        def x_off(c):
            h = c // sub_per_h
            return h * b_total + wb + (c % sub_per_h) * CHUNK

        pltpu.async_copy(x_hbm.at[pl.ds(x_off(0), CHUNK)], idx_v[0], sem_i[0])
        pltpu.make_async_copy(
            x_hbm.at[pl.ds(x_off(0), CHUNK)], idx_v[0], sem_i[0]
        ).wait()
        pltpu.async_copy(table_hbm.at[idx_v[0]], rows[0], sem_g[0])
        pltpu.async_copy(x_hbm.at[pl.ds(x_off(1), CHUNK)], idx_v[1], sem_i[1])

        @pl.loop(0, n_chunks, step=NBUF)
        def _chunk_loop(c0):
            for b in range(NBUF):
                c = c0 + b
                nb = (b + 1) % NBUF
                nxt = c + 1
                h = c // sub_per_h
                bb = wb + (c % sub_per_h) * CHUNK

                @pl.when(nxt < n_chunks)
                def _issue_next_gather():
                    pltpu.make_async_copy(
                        x_hbm.at[pl.ds(x_off(nxt), CHUNK)], idx_v[nb], sem_i[nb]
                    ).wait()
                    pltpu.async_copy(
                        table_hbm.at[idx_v[nb]], rows[nb], sem_g[nb]
                    )

                pltpu.make_async_copy(
                    table_hbm.at[idx_v[b]], rows[b], sem_g[b]
                ).wait()

                @pl.when(c + 2 < n_chunks)
                def _prefetch_idx():
                    pltpu.async_copy(
                        x_hbm.at[pl.ds(x_off(c + 2), CHUNK)],
                        idx_v[b],
                        sem_i[b],
                    )

                @pl.when(c > 0)
                def _drain_prev_tiles():
                    for t in range(16):
                        i, j = divmod(t, 2)
                        pltpu.make_async_copy(
                            tbuf.at[t],
                            out_hbm.at[
                                h, pl.ds(8 * i, 8), pl.ds(bb + 128 * j, 128)
                            ],
                            sem_t,
                        ).wait()

                @pl.loop(0, LANES)
                def _g16(g16):
                    b0l = g16 * LANES
                    j = g16 // 8
                    row_idx = biota + b0l
                    lane_idx = biota + (b0l - j * 128)
                    for dg in range(4):
                        d0 = dg * LANES
                        for k in range(LANES):
                            vals = plsc.load_gather(
                                rows[b], [row_idx, e_k[k] + d0]
                            )
                            plsc.store_scatter(
                                tbuf,
                                [r_k[k] + (4 * dg + j), s_k[k], lane_idx],
                                vals * SCALE,
                            )

                for t in range(16):
                    i, j = divmod(t, 2)
                    pltpu.async_copy(
                        tbuf.at[t],
                        out_hbm.at[
                            h, pl.ds(8 * i, 8), pl.ds(bb + 128 * j, 128)
                        ],
                        sem_t,
                    )

        lastc = n_chunks - 1
        lh = lastc // sub_per_h
        lbb = wb + (lastc % sub_per_h) * CHUNK
        for t in range(16):
            i, j = divmod(t, 2)
            pltpu.make_async_copy(
                tbuf.at[t],
                out_hbm.at[lh, pl.ds(8 * i, 8), pl.ds(lbb + 128 * j, 128)],
                sem_t,
            ).wait()

    return k(x_flat, table_p)


def kernel(x, table):
    b, h = x.shape
    x_flat = x.T.reshape(-1)
    table_p = jnp.pad(table, ((0, 0), (0, PAD_D - EMB_D)))
    out = _lookup(x_flat, table_p, b, h)
    return out.transpose(2, 0, 1)


# speedup vs baseline: 1.8733x; 1.0219x over previous
import functools

import jax
import jax.numpy as jnp
from jax import lax
from jax.experimental import pallas as pl
from jax.experimental.pallas import tpu as pltpu
from jax.experimental.pallas import tpu_sc as plsc

EMB_D = 64
PAD_D = 128
SCALE = float(EMB_D) ** 0.5
LANES = 16
NBUF = 2
CHUNK = 256


@functools.partial(jax.jit, static_argnames=("b_total", "h_total"))
def _lookup(x_flat, table_p, b_total, h_total):
    info = plsc.get_sparse_core_info()
    nw = info.num_cores * info.num_subcores
    b_per_w = b_total // nw
    sub_per_h = b_per_w // CHUNK
    n_chunks = h_total * sub_per_h
    assert b_per_w % CHUNK == 0 and b_total % nw == 0

    mesh = plsc.VectorSubcoreMesh(core_axis_name="c", subcore_axis_name="s")

    @functools.partial(
        pl.kernel,
        mesh=mesh,
        out_type=jax.ShapeDtypeStruct((h_total, EMB_D, b_total), jnp.float32),
        scratch_types=[
            [pltpu.VMEM((CHUNK,), jnp.int32) for _ in range(NBUF)],
            [pltpu.VMEM((CHUNK, PAD_D), jnp.float32) for _ in range(NBUF)],
            pltpu.VMEM((128, 128), jnp.float32),
            [pltpu.SemaphoreType.DMA for _ in range(NBUF)],
            pltpu.SemaphoreType.DMA,
            [pltpu.SemaphoreType.DMA for _ in range(NBUF)],
        ],
        compiler_params=pltpu.CompilerParams(
            use_tc_tiling_on_sc=True, needs_layout_passes=False
        ),
    )
    def k(x_hbm, table_hbm, out_hbm, idx_v, rows, tbuf, sem_g, sem_t, sem_i):
        wid = lax.axis_index("s") * info.num_cores + lax.axis_index("c")
        wb = wid * b_per_w
        biota = lax.iota(jnp.int32, LANES)
        e_k = [(biota + k) & 15 for k in range(LANES)]
        rs_k = [e + (e & 8) for e in e_k]

        def x_off(c):
            h = c // sub_per_h
            return h * b_total + wb + (c % sub_per_h) * CHUNK

        pltpu.async_copy(x_hbm.at[pl.ds(x_off(0), CHUNK)], idx_v[0], sem_i[0])
        pltpu.make_async_copy(
            x_hbm.at[pl.ds(x_off(0), CHUNK)], idx_v[0], sem_i[0]
        ).wait()
        pltpu.async_copy(table_hbm.at[idx_v[0]], rows[0], sem_g[0])
        pltpu.async_copy(x_hbm.at[pl.ds(x_off(1), CHUNK)], idx_v[1], sem_i[1])

        @pl.loop(0, n_chunks, step=NBUF)
        def _chunk_loop(c0):
            for b in range(NBUF):
                c = c0 + b
                nb = (b + 1) % NBUF
                nxt = c + 1
                h = c // sub_per_h
                bb = wb + (c % sub_per_h) * CHUNK

                @pl.when(nxt < n_chunks)
                def _issue_next_gather():
                    pltpu.make_async_copy(
                        x_hbm.at[pl.ds(x_off(nxt), CHUNK)], idx_v[nb], sem_i[nb]
                    ).wait()
                    pltpu.async_copy(
                        table_hbm.at[idx_v[nb]], rows[nb], sem_g[nb]
                    )

                pltpu.make_async_copy(
                    table_hbm.at[idx_v[b]], rows[b], sem_g[b]
                ).wait()

                @pl.when(c + 2 < n_chunks)
                def _prefetch_idx():
                    pltpu.async_copy(
                        x_hbm.at[pl.ds(x_off(c + 2), CHUNK)],
                        idx_v[b],
                        sem_i[b],
                    )

                @pl.when(c > 0)
                def _drain_prev_tiles():
                    for t in range(16):
                        i, j = divmod(t, 2)
                        pltpu.make_async_copy(
                            tbuf.at[pl.ds(8 * t, 8), pl.ds(0, 128)],
                            out_hbm.at[
                                h, pl.ds(8 * i, 8), pl.ds(bb + 128 * j, 128)
                            ],
                            sem_t,
                        ).wait()

                @pl.loop(0, LANES)
                def _g16(g16):
                    b0l = g16 * LANES
                    j = g16 // 8
                    row_idx = biota + b0l
                    lane_idx = biota + (b0l - j * 128)
                    for dg in range(4):
                        d0 = dg * LANES
                        for k in range(LANES):
                            vals = plsc.load_gather(
                                rows[b], [row_idx, e_k[k] + d0]
                            )
                            plsc.store_scatter(
                                tbuf,
                                [rs_k[k] + (32 * dg + 8 * j), lane_idx],
                                vals * SCALE,
                            )

                for t in range(16):
                    i, j = divmod(t, 2)
                    pltpu.async_copy(
                        tbuf.at[pl.ds(8 * t, 8), pl.ds(0, 128)],
                        out_hbm.at[
                            h, pl.ds(8 * i, 8), pl.ds(bb + 128 * j, 128)
                        ],
                        sem_t,
                    )

        lastc = n_chunks - 1
        lh = lastc // sub_per_h
        lbb = wb + (lastc % sub_per_h) * CHUNK
        for t in range(16):
            i, j = divmod(t, 2)
            pltpu.make_async_copy(
                tbuf.at[pl.ds(8 * t, 8), pl.ds(0, 128)],
                out_hbm.at[lh, pl.ds(8 * i, 8), pl.ds(lbb + 128 * j, 128)],
                sem_t,
            ).wait()

    return k(x_flat, table_p)


def kernel(x, table):
    b, h = x.shape
    x_flat = x.T.reshape(-1)
    table_p = jnp.pad(table, ((0, 0), (0, PAD_D - EMB_D)))
    out = _lookup(x_flat, table_p, b, h)
    return out.transpose(2, 0, 1)
